# baseline (speedup 1.0000x reference)
import functools

import jax
import jax.numpy as jnp
from jax import lax
from jax.experimental import pallas as pl
from jax.experimental.pallas import tpu as pltpu
from jax.experimental.pallas import tpu_sc as plsc

DH = 128
NTILES = 32
BK = 128
RB = 1024
GB = RB // DH
SC0_FRAC = 0.72


def _sc_pass_body(nb0, nb1, npad, table, gat_idx, sca_idx, out, out_h,
                  gidx_v, sidx_v, rows_v, hist_v, acc_sh,
                  sem, semig, semis, sems2):
    cid = lax.axis_index("c")
    sid = lax.axis_index("s")
    base = jnp.where(cid == 0, sid * nb0, 16 * nb0 + sid * nb1)
    nbc = jnp.where(cid == 0, nb0, nb1)
    rows_per_tile = npad // 16
    zero16 = jnp.zeros((16,), jnp.float32)
    ones16 = jnp.full((16,), 1.0, jnp.float32)

    def zrow(i, c):
        for k in range(DH // 16):
            rows_v[0, i, pl.ds(k * 16, 16)] = zero16
        return c

    lax.fori_loop(0, BK, zrow, 0)

    def zhist(i, c):
        hist_v[pl.ds(i * 16, 16)] = zero16
        return c

    lax.fori_loop(0, npad // 16, zhist, 0)
    for k in range(rows_per_tile // BK):
        pltpu.sync_copy(rows_v.at[0],
                        acc_sh.at[pl.ds(sid * rows_per_tile + k * BK, BK)])
    plsc.subcore_barrier()

    pltpu.sync_copy(gat_idx.at[base], gidx_v.at[0])
    pltpu.sync_copy(sca_idx.at[base], sidx_v.at[0])
    pltpu.sync_copy(gat_idx.at[base + 1], gidx_v.at[1])
    pltpu.sync_copy(sca_idx.at[base + 1], sidx_v.at[1])
    pltpu.async_copy(table.at[gidx_v.at[0]], rows_v.at[0], sem)

    def one_block(j, u):
        rs = u % 2
        rs2 = 1 - rs
        is_ = u % 4

        @pl.when(j < nbc)
        def _():
            pltpu.make_async_copy(table.at[gidx_v.at[is_]], rows_v.at[rs],
                                  sem).wait()

            @pl.when(j > 0)
            def _():
                pltpu.make_async_copy(rows_v.at[rs2],
                                      acc_sh.at[sidx_v.at[(u - 1) % 4]],
                                      sems2).wait()

            @pl.when(j + 1 < nbc)
            def _():
                pltpu.async_copy(table.at[gidx_v.at[(u + 1) % 4]],
                                 rows_v.at[rs2], sem)

            @pl.when(j + 2 < nbc)
            def _():
                pltpu.async_copy(gat_idx.at[base + j + 2],
                                 gidx_v.at[(u + 2) % 4], semig)
                pltpu.async_copy(sca_idx.at[base + j + 2],
                                 sidx_v.at[(u + 2) % 4], semis)

            for k in range(BK // 16):
                idx = sidx_v[is_, pl.ds(k * 16, 16)]
                plsc.addupdate_scatter(hist_v, [idx], ones16)
            pltpu.async_copy(rows_v.at[rs], acc_sh.at[sidx_v.at[is_]], sems2,
                             add=True)

            @pl.when(j + 2 < nbc)
            def _():
                pltpu.make_async_copy(gat_idx.at[base + j + 2],
                                      gidx_v.at[(u + 2) % 4], semig).wait()
                pltpu.make_async_copy(sca_idx.at[base + j + 2],
                                      sidx_v.at[(u + 2) % 4], semis).wait()

    def quad(q, c):
        for u in range(4):
            one_block(q * 4 + u, u)
        return c

    lax.fori_loop(0, (jnp.maximum(nbc, 1) + 3) // 4, quad, 0)

    @pl.when(nbc > 0)
    def _():
        last = nbc - 1
        pltpu.make_async_copy(rows_v.at[last % 2],
                              acc_sh.at[sidx_v.at[last % 4]], sems2).wait()
    pltpu.sync_copy(hist_v, out_h.at[cid, sid])
    plsc.subcore_barrier()
    pltpu.sync_copy(acc_sh.at[pl.ds(sid * rows_per_tile, rows_per_tile)],
                    out.at[cid, pl.ds(sid * rows_per_tile, rows_per_tile)])


def _make_sc_pass(nb0, nb1, npad):
    mesh = plsc.VectorSubcoreMesh(core_axis_name="c", subcore_axis_name="s")
    return pl.kernel(
        functools.partial(_sc_pass_body, nb0, nb1, npad),
        out_type=(jax.ShapeDtypeStruct((2, npad, DH), jnp.float32),
                  jax.ShapeDtypeStruct((2, 16, npad), jnp.float32)),
        mesh=mesh,
        compiler_params=pltpu.CompilerParams(needs_layout_passes=False),
        scratch_types=[
            pltpu.VMEM((4, BK), jnp.int32),
            pltpu.VMEM((4, BK), jnp.int32),
            pltpu.VMEM((2, BK, DH), jnp.float32),
            pltpu.VMEM((npad,), jnp.float32),
            pltpu.VMEM_SHARED((npad, DH), jnp.float32),
            pltpu.SemaphoreType.DMA,
            pltpu.SemaphoreType.DMA,
            pltpu.SemaphoreType.DMA,
            pltpu.SemaphoreType.DMA,
        ],
    )


def _mm_body(x_ref, w_ref, o_ref):
    o_ref[...] = lax.dot_general(x_ref[...], w_ref[...], (((1,), (1,)), ((), ())),
                                 preferred_element_type=jnp.float32)


def _inv_cnt_col(h_ref):
    cnt = jnp.sum(h_ref[...], axis=0)
    eye = jnp.where(
        lax.broadcasted_iota(jnp.int32, (DH, DH), 0)
        == lax.broadcasted_iota(jnp.int32, (DH, DH), 1), 1.0, 0.0)
    cols = [lax.dot_general(eye, cnt[c:c + 1, :], (((1,), (1,)), ((), ())),
                            preferred_element_type=jnp.float32)
            for c in range(GB)]
    cntcol = jnp.concatenate(cols, axis=0)
    return jnp.where(cntcol > 0.0, 1.0 / jnp.where(cntcol > 0.0, cntcol, 1.0), 0.0)


def _mid_body(s_ref, h_ref, o_ref):
    o_ref[...] = _inv_cnt_col(h_ref) * (s_ref[0] + s_ref[1])


def _fuse_body(s_ref, h_ref, b_ref, w_ref, o_ref):
    h = jnp.maximum(_inv_cnt_col(h_ref) * (s_ref[0] + s_ref[1]) + b_ref[...], 0.0)
    o_ref[...] = lax.dot_general(h, w_ref[...], (((1,), (1,)), ((), ())),
                                 preferred_element_type=jnp.float32)


def _final_body(n, grid, s_ref, h_ref, b_ref, o_ref):
    i = pl.program_id(0)
    h = jnp.maximum(_inv_cnt_col(h_ref) * (s_ref[0] + s_ref[1]) + b_ref[...], 0.0)
    row = lax.broadcasted_iota(jnp.int32, h.shape, 0) + i * h.shape[0]
    h = jnp.where(row < n, h, 0.0)
    ps = jnp.sum(h, axis=0, keepdims=True)
    prev = jnp.where(i == 0, 0.0, o_ref[...])
    o_ref[...] = (prev + ps) * jnp.where(i == grid - 1, 1.0 / n, 1.0)


def kernel(x, edge_index, W1, b1, W2, b2):
    x = x.astype(jnp.float32)
    ei = edge_index.astype(jnp.int32)
    n = x.shape[0]
    e = ei.shape[1]
    npad = ((n + 1 + 2047) // 2048) * 2048
    ssum = (e + 16 * BK - 1) // (16 * BK)
    nb0 = min(ssum - 1, max(1, round(ssum * SC0_FRAC)))
    nb1 = ssum - nb0
    epad = 16 * ssum * BK
    grid = npad // RB
    g = npad // DH
    dump = n

    node_idx, edge_idx = ei[0], ei[1]
    zpad = jnp.zeros((epad - e,), jnp.int32)
    dpad = jnp.full((epad - e,), dump, jnp.int32)
    nodes_g = jnp.concatenate([node_idx, zpad]).reshape(16 * ssum, BK)
    edges_s = jnp.concatenate([edge_idx, dpad]).reshape(16 * ssum, BK)
    edges_g = jnp.concatenate([edge_idx, zpad]).reshape(16 * ssum, BK)
    nodes_s = jnp.concatenate([node_idx, dpad]).reshape(16 * ssum, BK)

    x_p = jnp.pad(x, ((0, npad - n), (0, 0)))
    W1f = W1.astype(jnp.float32)
    W2f = W2.astype(jnp.float32)
    b1r = b1.astype(jnp.float32).reshape(1, DH)
    b2r = b2.astype(jnp.float32).reshape(1, DH)

    f32 = jnp.float32
    s_spec = pl.BlockSpec((2, RB, DH), lambda i: (0, i, 0))
    h_spec = pl.BlockSpec((NTILES, GB, DH), lambda i: (0, i, 0))
    r_spec = pl.BlockSpec((RB, DH), lambda i: (i, 0))
    w_spec = pl.BlockSpec((DH, DH), lambda i: (0, 0))
    b_spec = pl.BlockSpec((1, DH), lambda i: (0, 0))
    nd_shape = jax.ShapeDtypeStruct((npad, DH), f32)

    k_in = pl.pallas_call(
        _mm_body, grid=(grid,),
        in_specs=[r_spec, w_spec], out_specs=r_spec, out_shape=nd_shape)
    k_mid = pl.pallas_call(
        _mid_body, grid=(grid,),
        in_specs=[s_spec, h_spec], out_specs=r_spec, out_shape=nd_shape)
    k_fuse = pl.pallas_call(
        _fuse_body, grid=(grid,),
        in_specs=[s_spec, h_spec, b_spec, w_spec],
        out_specs=r_spec, out_shape=nd_shape)
    k_final = pl.pallas_call(
        functools.partial(_final_body, n, grid), grid=(grid,),
        in_specs=[s_spec, h_spec, b_spec],
        out_specs=pl.BlockSpec((1, DH), lambda i: (0, 0)),
        out_shape=jax.ShapeDtypeStruct((1, DH), f32))
    sc_pass = _make_sc_pass(nb0, nb1, npad)

    def _h(hraw):
        return hraw.reshape(NTILES, g, DH)

    y1 = k_in(x_p, W1f)
    s1, he1 = sc_pass(y1, nodes_g, edges_s)
    s1b = k_mid(s1, _h(he1))
    s2, hn1 = sc_pass(s1b, edges_g, nodes_s)
    y2 = k_fuse(s2, _h(hn1), b1r, W2f)
    s3, he2 = sc_pass(y2, nodes_g, edges_s)
    s3b = k_mid(s3, _h(he2))
    s4, hn2 = sc_pass(s3b, edges_g, nodes_s)
    return k_final(s4, _h(hn2), b2r)


# speedup vs baseline: 1.0472x; 1.0472x over previous
import functools

import jax
import jax.numpy as jnp
from jax import lax
from jax.experimental import pallas as pl
from jax.experimental.pallas import tpu as pltpu
from jax.experimental.pallas import tpu_sc as plsc

DH = 128
NTILES = 32
BK = 128
RB = 1024
GB = RB // DH
SC0_FRAC = 0.68


def _sc_pass_body(nb0, nb1, npad, table, gat_idx, sca_idx, out, out_h,
                  gidx_v, sidx_v, rows_v, hist_v, acc_sh,
                  sem, semig, semis, sems2):
    cid = lax.axis_index("c")
    sid = lax.axis_index("s")
    base = jnp.where(cid == 0, sid * nb0, 16 * nb0 + sid * nb1)
    nbc = jnp.where(cid == 0, nb0, nb1)
    rows_per_tile = npad // 16
    zero16 = jnp.zeros((16,), jnp.float32)
    ones16 = jnp.full((16,), 1.0, jnp.float32)

    def zrow(i, c):
        for k in range(DH // 16):
            rows_v[0, i, pl.ds(k * 16, 16)] = zero16
        return c

    lax.fori_loop(0, BK, zrow, 0)

    def zhist(i, c):
        hist_v[pl.ds(i * 16, 16)] = zero16
        return c

    lax.fori_loop(0, npad // 16, zhist, 0)
    for k in range(rows_per_tile // BK):
        pltpu.sync_copy(rows_v.at[0],
                        acc_sh.at[pl.ds(sid * rows_per_tile + k * BK, BK)])
    plsc.subcore_barrier()

    pltpu.sync_copy(gat_idx.at[base], gidx_v.at[0])
    pltpu.sync_copy(sca_idx.at[base], sidx_v.at[0])
    pltpu.sync_copy(gat_idx.at[base + 1], gidx_v.at[1])
    pltpu.sync_copy(sca_idx.at[base + 1], sidx_v.at[1])
    pltpu.async_copy(table.at[gidx_v.at[0, 0]],
                     rows_v.at[0, pl.ds(0, BK // 2)], sem)
    pltpu.async_copy(table.at[gidx_v.at[0, 1]],
                     rows_v.at[0, pl.ds(BK // 2, BK // 2)], sem)

    def one_block(j, u):
        rs = u % 2
        rs2 = 1 - rs
        is_ = u % 4

        @pl.when(j < nbc)
        def _():
            pltpu.make_async_copy(table.at[gidx_v.at[is_, 0]],
                                  rows_v.at[rs, pl.ds(0, BK // 2)],
                                  sem).wait()
            pltpu.make_async_copy(table.at[gidx_v.at[is_, 1]],
                                  rows_v.at[rs, pl.ds(BK // 2, BK // 2)],
                                  sem).wait()

            @pl.when(j > 0)
            def _():
                pltpu.make_async_copy(rows_v.at[rs2],
                                      acc_sh.at[sidx_v.at[(u - 1) % 4]],
                                      sems2).wait()

            @pl.when(j + 1 < nbc)
            def _():
                pltpu.async_copy(table.at[gidx_v.at[(u + 1) % 4, 0]],
                                 rows_v.at[rs2, pl.ds(0, BK // 2)], sem)
                pltpu.async_copy(table.at[gidx_v.at[(u + 1) % 4, 1]],
                                 rows_v.at[rs2, pl.ds(BK // 2, BK // 2)], sem)

            @pl.when(j + 2 < nbc)
            def _():
                pltpu.async_copy(gat_idx.at[base + j + 2],
                                 gidx_v.at[(u + 2) % 4], semig)
                pltpu.async_copy(sca_idx.at[base + j + 2],
                                 sidx_v.at[(u + 2) % 4], semis)

            for k in range(BK // 16):
                idx = sidx_v[is_, pl.ds(k * 16, 16)]
                plsc.addupdate_scatter(hist_v, [idx], ones16)
            pltpu.async_copy(rows_v.at[rs], acc_sh.at[sidx_v.at[is_]], sems2,
                             add=True)

            @pl.when(j + 2 < nbc)
            def _():
                pltpu.make_async_copy(gat_idx.at[base + j + 2],
                                      gidx_v.at[(u + 2) % 4], semig).wait()
                pltpu.make_async_copy(sca_idx.at[base + j + 2],
                                      sidx_v.at[(u + 2) % 4], semis).wait()

    def quad(q, c):
        for u in range(4):
            one_block(q * 4 + u, u)
        return c

    lax.fori_loop(0, (jnp.maximum(nbc, 1) + 3) // 4, quad, 0)

    @pl.when(nbc > 0)
    def _():
        last = nbc - 1
        pltpu.make_async_copy(rows_v.at[last % 2],
                              acc_sh.at[sidx_v.at[last % 4]], sems2).wait()
    pltpu.sync_copy(hist_v, out_h.at[cid, sid])
    plsc.subcore_barrier()
    pltpu.sync_copy(acc_sh.at[pl.ds(sid * rows_per_tile, rows_per_tile)],
                    out.at[cid, pl.ds(sid * rows_per_tile, rows_per_tile)])


def _make_sc_pass(nb0, nb1, npad):
    mesh = plsc.VectorSubcoreMesh(core_axis_name="c", subcore_axis_name="s")
    return pl.kernel(
        functools.partial(_sc_pass_body, nb0, nb1, npad),
        out_type=(jax.ShapeDtypeStruct((2, npad, DH), jnp.float32),
                  jax.ShapeDtypeStruct((2, 16, npad), jnp.float32)),
        mesh=mesh,
        compiler_params=pltpu.CompilerParams(needs_layout_passes=False),
        scratch_types=[
            pltpu.VMEM((4, 2, BK // 2), jnp.int32),
            pltpu.VMEM((4, BK), jnp.int32),
            pltpu.VMEM((2, BK, DH), jnp.float32),
            pltpu.VMEM((npad,), jnp.float32),
            pltpu.VMEM_SHARED((npad, DH), jnp.float32),
            pltpu.SemaphoreType.DMA,
            pltpu.SemaphoreType.DMA,
            pltpu.SemaphoreType.DMA,
            pltpu.SemaphoreType.DMA,
        ],
    )


def _mm_body(x_ref, w_ref, o_ref):
    o_ref[...] = lax.dot_general(x_ref[...], w_ref[...], (((1,), (1,)), ((), ())),
                                 preferred_element_type=jnp.float32)


def _inv_cnt_col(h_ref):
    cnt = jnp.sum(h_ref[...], axis=0)
    eye = jnp.where(
        lax.broadcasted_iota(jnp.int32, (DH, DH), 0)
        == lax.broadcasted_iota(jnp.int32, (DH, DH), 1), 1.0, 0.0)
    cols = [lax.dot_general(eye, cnt[c:c + 1, :], (((1,), (1,)), ((), ())),
                            preferred_element_type=jnp.float32)
            for c in range(GB)]
    cntcol = jnp.concatenate(cols, axis=0)
    return jnp.where(cntcol > 0.0, 1.0 / jnp.where(cntcol > 0.0, cntcol, 1.0), 0.0)


def _mid_body(s_ref, h_ref, o_ref):
    o_ref[...] = _inv_cnt_col(h_ref) * (s_ref[0] + s_ref[1])


def _fuse_body(s_ref, h_ref, b_ref, w_ref, o_ref):
    h = jnp.maximum(_inv_cnt_col(h_ref) * (s_ref[0] + s_ref[1]) + b_ref[...], 0.0)
    o_ref[...] = lax.dot_general(h, w_ref[...], (((1,), (1,)), ((), ())),
                                 preferred_element_type=jnp.float32)


def _final_body(n, grid, s_ref, h_ref, b_ref, o_ref):
    i = pl.program_id(0)
    h = jnp.maximum(_inv_cnt_col(h_ref) * (s_ref[0] + s_ref[1]) + b_ref[...], 0.0)
    row = lax.broadcasted_iota(jnp.int32, h.shape, 0) + i * h.shape[0]
    h = jnp.where(row < n, h, 0.0)
    ps = jnp.sum(h, axis=0, keepdims=True)
    prev = jnp.where(i == 0, 0.0, o_ref[...])
    o_ref[...] = (prev + ps) * jnp.where(i == grid - 1, 1.0 / n, 1.0)


def kernel(x, edge_index, W1, b1, W2, b2):
    x = x.astype(jnp.float32)
    ei = edge_index.astype(jnp.int32)
    n = x.shape[0]
    e = ei.shape[1]
    npad = ((n + 1 + 2047) // 2048) * 2048
    ssum = (e + 16 * BK - 1) // (16 * BK)
    nb0 = min(ssum - 1, max(1, round(ssum * SC0_FRAC)))
    nb1 = ssum - nb0
    epad = 16 * ssum * BK
    grid = npad // RB
    g = npad // DH
    dump = n

    node_idx, edge_idx = ei[0], ei[1]
    zpad = jnp.zeros((epad - e,), jnp.int32)
    dpad = jnp.full((epad - e,), dump, jnp.int32)
    nodes_g = jnp.concatenate([node_idx, zpad]).reshape(16 * ssum, 2, BK // 2)
    edges_s = jnp.concatenate([edge_idx, dpad]).reshape(16 * ssum, BK)
    edges_g = jnp.concatenate([edge_idx, zpad]).reshape(16 * ssum, 2, BK // 2)
    nodes_s = jnp.concatenate([node_idx, dpad]).reshape(16 * ssum, BK)

    x_p = jnp.pad(x, ((0, npad - n), (0, 0)))
    W1f = W1.astype(jnp.float32)
    W2f = W2.astype(jnp.float32)
    b1r = b1.astype(jnp.float32).reshape(1, DH)
    b2r = b2.astype(jnp.float32).reshape(1, DH)

    f32 = jnp.float32
    s_spec = pl.BlockSpec((2, RB, DH), lambda i: (0, i, 0))
    h_spec = pl.BlockSpec((NTILES, GB, DH), lambda i: (0, i, 0))
    r_spec = pl.BlockSpec((RB, DH), lambda i: (i, 0))
    w_spec = pl.BlockSpec((DH, DH), lambda i: (0, 0))
    b_spec = pl.BlockSpec((1, DH), lambda i: (0, 0))
    nd_shape = jax.ShapeDtypeStruct((npad, DH), f32)

    k_in = pl.pallas_call(
        _mm_body, grid=(grid,),
        in_specs=[r_spec, w_spec], out_specs=r_spec, out_shape=nd_shape)
    k_mid = pl.pallas_call(
        _mid_body, grid=(grid,),
        in_specs=[s_spec, h_spec], out_specs=r_spec, out_shape=nd_shape)
    k_fuse = pl.pallas_call(
        _fuse_body, grid=(grid,),
        in_specs=[s_spec, h_spec, b_spec, w_spec],
        out_specs=r_spec, out_shape=nd_shape)
    k_final = pl.pallas_call(
        functools.partial(_final_body, n, grid), grid=(grid,),
        in_specs=[s_spec, h_spec, b_spec],
        out_specs=pl.BlockSpec((1, DH), lambda i: (0, 0)),
        out_shape=jax.ShapeDtypeStruct((1, DH), f32))
    sc_pass = _make_sc_pass(nb0, nb1, npad)

    def _h(hraw):
        return hraw.reshape(NTILES, g, DH)

    y1 = k_in(x_p, W1f)
    s1, he1 = sc_pass(y1, nodes_g, edges_s)
    s1b = k_mid(s1, _h(he1))
    s2, hn1 = sc_pass(s1b, edges_g, nodes_s)
    y2 = k_fuse(s2, _h(hn1), b1r, W2f)
    s3, he2 = sc_pass(y2, nodes_g, edges_s)
    s3b = k_mid(s3, _h(he2))
    s4, hn2 = sc_pass(s3b, edges_g, nodes_s)
    return k_final(s4, _h(hn2), b2r)
